# baseline (speedup 1.0000x reference)
import jax
import jax.numpy as jnp
from jax import lax
from jax.experimental import pallas as pl
from jax.experimental.pallas import tpu as pltpu

_B = 16
_TB = 2048
_VB = 8192


def _make_body(nb):
    def _body(bq_ref, batch2_ref, batch_ref, se_hbm, se_ref, W1_ref, W2_ref,
              Wqb_ref, bias_ref, W3_ref, b3_ref, iw_ref, z_ref,
              acc_ref, sh_ref, vn_ref, sem):
        i = pl.program_id(0)

        @pl.when(i == 0)
        def _head():
            b2d = batch2_ref[...]
            copies = []
            for s in range(_B):
                cnt = jnp.sum((b2d <= s).astype(jnp.int32))
                idx = jnp.maximum(cnt - 1, 0)
                cp = pltpu.make_async_copy(
                    se_hbm.at[pl.ds(idx, 1)], vn_ref.at[pl.ds(s, 1)], sem)
                cp.start()
                copies.append(cp)
            for cp in copies:
                cp.wait()

        @pl.when(i < nb)
        def _token():
            se = se_ref[...]
            tb = se.shape[0]
            brow = batch_ref[0]
            seg_iota = lax.broadcasted_iota(jnp.int32, (_B, tb), 0)
            ohT = (jnp.broadcast_to(brow, (_B, tb)) == seg_iota
                   ).astype(jnp.bfloat16)
            c = lax.dot_general(vn_ref[...], W1_ref[...],
                                (((1,), (1,)), ((), ())),
                                preferred_element_type=jnp.float32)
            c = (c + bias_ref[...]).astype(jnp.bfloat16)
            m = lax.dot_general(se, W2_ref[...], (((1,), (1,)), ((), ())),
                                preferred_element_type=jnp.float32)
            g = lax.dot_general(ohT, c, (((0,), (0,)), ((), ())),
                                preferred_element_type=jnp.float32)
            h = jax.nn.sigmoid(m + g).astype(jnp.bfloat16)
            alphaT = lax.dot_general(Wqb_ref[...], h, (((1,), (1,)), ((), ())),
                                     preferred_element_type=jnp.float32)
            wT = (ohT.astype(jnp.float32) * (alphaT + bq_ref[0])
                  ).astype(jnp.bfloat16)
            part = lax.dot_general(wT, se.astype(jnp.bfloat16),
                                   (((1,), (0,)), ((), ())),
                                   preferred_element_type=jnp.float32)

            @pl.when(i == 0)
            def _():
                acc_ref[...] = jnp.zeros_like(acc_ref)

            acc_ref[...] += part

            @pl.when(i == nb - 1)
            def _():
                cat = jnp.concatenate([vn_ref[...], acc_ref[...]], axis=1)
                sh_ref[...] = lax.dot_general(
                    cat, W3_ref[...], (((1,), (1,)), ((), ())),
                    preferred_element_type=jnp.float32) + b3_ref[...]

        @pl.when(i >= nb)
        def _vocab():
            z_ref[...] = lax.dot_general(sh_ref[...], iw_ref[...],
                                         (((1,), (1,)), ((), ())),
                                         preferred_element_type=jnp.float32)

    return _body


def kernel(session_embedding, batch, item_weight, W1, b1, W2, b2, Wq, bq, W3, b3):
    T, H = session_embedding.shape
    V = item_weight.shape[0]
    batch32 = batch.astype(jnp.int32)

    nb = T // _TB
    nv = pl.cdiv(V, _VB)
    batch2 = batch32.reshape(T // 128, 128)
    batch3 = batch32.reshape(nb, 1, _TB)
    bias = (b1 + b2).reshape(1, H)
    Wqb = jnp.broadcast_to(Wq, (_B, H))

    tok = lambda i: jnp.minimum(i, nb - 1)
    voc = lambda i: jnp.maximum(i - nb, 0)

    z = pl.pallas_call(
        _make_body(nb),
        grid=(nb + nv,),
        in_specs=[
            pl.BlockSpec(memory_space=pltpu.SMEM),
            pl.BlockSpec((T // 128, 128), lambda i: (0, 0)),
            pl.BlockSpec((1, 1, _TB), lambda i: (tok(i), 0, 0)),
            pl.BlockSpec(memory_space=pl.ANY),
            pl.BlockSpec((_TB, H), lambda i: (tok(i), 0)),
            pl.BlockSpec((H, H), lambda i: (0, 0)),
            pl.BlockSpec((H, H), lambda i: (0, 0)),
            pl.BlockSpec((_B, H), lambda i: (0, 0)),
            pl.BlockSpec((1, H), lambda i: (0, 0)),
            pl.BlockSpec((H, 2 * H), lambda i: (0, 0)),
            pl.BlockSpec((1, H), lambda i: (0, 0)),
            pl.BlockSpec((_VB, H), lambda i: (voc(i), 0)),
        ],
        out_specs=pl.BlockSpec((_B, _VB), lambda i: (0, voc(i))),
        out_shape=jax.ShapeDtypeStruct((_B, V), jnp.float32),
        scratch_shapes=[pltpu.VMEM((_B, H), jnp.float32),
                        pltpu.VMEM((_B, H), jnp.float32),
                        pltpu.VMEM((_B, H), jnp.float32),
                        pltpu.SemaphoreType.DMA],
    )(bq, batch2, batch3, session_embedding, session_embedding, W1, W2,
      Wqb, bias, W3, b3.reshape(1, H), item_weight)
    return z


# speedup vs baseline: 1.0604x; 1.0604x over previous
import jax
import jax.numpy as jnp
from jax import lax
from jax.experimental import pallas as pl
from jax.experimental.pallas import tpu as pltpu

_B = 16
_TB = 8192
_VB = 8192


def _make_body(nb):
    def _body(bq_ref, batch2_ref, batch_ref, se_hbm, se_ref, W1_ref, W2_ref,
              Wqb_ref, bias_ref, W3_ref, b3_ref, iw_ref, z_ref,
              acc_ref, sh_ref, vn_ref, sem):
        i = pl.program_id(0)

        @pl.when(i == 0)
        def _head():
            b2d = batch2_ref[...]
            copies = []
            for s in range(_B):
                cnt = jnp.sum((b2d <= s).astype(jnp.int32))
                idx = jnp.maximum(cnt - 1, 0)
                cp = pltpu.make_async_copy(
                    se_hbm.at[pl.ds(idx, 1)], vn_ref.at[pl.ds(s, 1)], sem)
                cp.start()
                copies.append(cp)
            for cp in copies:
                cp.wait()

        @pl.when(i < nb)
        def _token():
            se = se_ref[...]
            tb = se.shape[0]
            brow = batch_ref[0]
            seg_iota = lax.broadcasted_iota(jnp.int32, (_B, tb), 0)
            ohT = (jnp.broadcast_to(brow, (_B, tb)) == seg_iota
                   ).astype(jnp.bfloat16)
            c = lax.dot_general(vn_ref[...], W1_ref[...],
                                (((1,), (1,)), ((), ())),
                                preferred_element_type=jnp.float32)
            c = (c + bias_ref[...]).astype(jnp.bfloat16)
            m = lax.dot_general(se, W2_ref[...], (((1,), (1,)), ((), ())),
                                preferred_element_type=jnp.float32)
            g = lax.dot_general(ohT, c, (((0,), (0,)), ((), ())),
                                preferred_element_type=jnp.float32)
            h = jax.nn.sigmoid(m + g).astype(jnp.bfloat16)
            alphaT = lax.dot_general(Wqb_ref[...], h, (((1,), (1,)), ((), ())),
                                     preferred_element_type=jnp.float32)
            wT = (ohT.astype(jnp.float32) * (alphaT + bq_ref[0])
                  ).astype(jnp.bfloat16)
            part = lax.dot_general(wT, se.astype(jnp.bfloat16),
                                   (((1,), (0,)), ((), ())),
                                   preferred_element_type=jnp.float32)

            @pl.when(i == 0)
            def _():
                acc_ref[...] = jnp.zeros_like(acc_ref)

            acc_ref[...] += part

            @pl.when(i == nb - 1)
            def _():
                cat = jnp.concatenate([vn_ref[...], acc_ref[...]], axis=1)
                sh_ref[...] = lax.dot_general(
                    cat, W3_ref[...], (((1,), (1,)), ((), ())),
                    preferred_element_type=jnp.float32) + b3_ref[...]

        @pl.when(i >= nb)
        def _vocab():
            z_ref[...] = lax.dot_general(sh_ref[...], iw_ref[...],
                                         (((1,), (1,)), ((), ())),
                                         preferred_element_type=jnp.float32)

    return _body


def kernel(session_embedding, batch, item_weight, W1, b1, W2, b2, Wq, bq, W3, b3):
    T, H = session_embedding.shape
    V = item_weight.shape[0]
    batch32 = batch.astype(jnp.int32)

    nb = T // _TB
    nv = pl.cdiv(V, _VB)
    batch2 = batch32.reshape(T // 128, 128)
    batch3 = batch32.reshape(nb, 1, _TB)
    bias = (b1 + b2).reshape(1, H)
    Wqb = jnp.broadcast_to(Wq, (_B, H))

    tok = lambda i: jnp.minimum(i, nb - 1)
    voc = lambda i: jnp.maximum(i - nb, 0)

    z = pl.pallas_call(
        _make_body(nb),
        grid=(nb + nv,),
        in_specs=[
            pl.BlockSpec(memory_space=pltpu.SMEM),
            pl.BlockSpec((T // 128, 128), lambda i: (0, 0)),
            pl.BlockSpec((1, 1, _TB), lambda i: (tok(i), 0, 0)),
            pl.BlockSpec(memory_space=pl.ANY),
            pl.BlockSpec((_TB, H), lambda i: (tok(i), 0)),
            pl.BlockSpec((H, H), lambda i: (0, 0)),
            pl.BlockSpec((H, H), lambda i: (0, 0)),
            pl.BlockSpec((_B, H), lambda i: (0, 0)),
            pl.BlockSpec((1, H), lambda i: (0, 0)),
            pl.BlockSpec((H, 2 * H), lambda i: (0, 0)),
            pl.BlockSpec((1, H), lambda i: (0, 0)),
            pl.BlockSpec((_VB, H), lambda i: (voc(i), 0)),
        ],
        out_specs=pl.BlockSpec((_B, _VB), lambda i: (0, voc(i))),
        out_shape=jax.ShapeDtypeStruct((_B, V), jnp.float32),
        scratch_shapes=[pltpu.VMEM((_B, H), jnp.float32),
                        pltpu.VMEM((_B, H), jnp.float32),
                        pltpu.VMEM((_B, H), jnp.float32),
                        pltpu.SemaphoreType.DMA],
    )(bq, batch2, batch3, session_embedding, session_embedding, W1, W2,
      Wqb, bias, W3, b3.reshape(1, H), item_weight)
    return z


# speedup vs baseline: 1.0673x; 1.0064x over previous
import jax
import jax.numpy as jnp
from jax import lax
from jax.experimental import pallas as pl
from jax.experimental.pallas import tpu as pltpu

_B = 16
_TB = 4096
_VB = 8192


def _make_body(nb):
    def _body(bq_ref, batch2_ref, batch_ref, se_hbm, se_ref, W1_ref, W2_ref,
              Wqb_ref, bias_ref, W3_ref, b3_ref, iw_ref, z_ref,
              acc_ref, sh_ref, vn_ref, sem):
        i = pl.program_id(0)

        @pl.when(i == 0)
        def _head():
            b2d = batch2_ref[...]
            copies = []
            for s in range(_B):
                cnt = jnp.sum((b2d <= s).astype(jnp.int32))
                idx = jnp.maximum(cnt - 1, 0)
                cp = pltpu.make_async_copy(
                    se_hbm.at[pl.ds(idx, 1)], vn_ref.at[pl.ds(s, 1)], sem)
                cp.start()
                copies.append(cp)
            for cp in copies:
                cp.wait()

        @pl.when(i < nb)
        def _token():
            se = se_ref[...]
            tb = se.shape[0]
            brow = batch_ref[0]
            seg_iota = lax.broadcasted_iota(jnp.int32, (_B, tb), 0)
            ohT = (jnp.broadcast_to(brow, (_B, tb)) == seg_iota
                   ).astype(jnp.bfloat16)
            c = lax.dot_general(vn_ref[...], W1_ref[...],
                                (((1,), (1,)), ((), ())),
                                preferred_element_type=jnp.float32)
            c = (c + bias_ref[...]).astype(jnp.bfloat16)
            m = lax.dot_general(se, W2_ref[...], (((1,), (1,)), ((), ())),
                                preferred_element_type=jnp.float32)
            g = lax.dot_general(ohT, c, (((0,), (0,)), ((), ())),
                                preferred_element_type=jnp.float32)
            h = (0.5 + 0.5 * jnp.tanh(0.5 * (m + g))).astype(jnp.bfloat16)
            alphaT = lax.dot_general(Wqb_ref[...], h, (((1,), (1,)), ((), ())),
                                     preferred_element_type=jnp.float32)
            wT = (ohT.astype(jnp.float32) * (alphaT + bq_ref[0])
                  ).astype(jnp.bfloat16)
            part = lax.dot_general(wT, se.astype(jnp.bfloat16),
                                   (((1,), (0,)), ((), ())),
                                   preferred_element_type=jnp.float32)

            @pl.when(i == 0)
            def _():
                acc_ref[...] = jnp.zeros_like(acc_ref)

            acc_ref[...] += part

            @pl.when(i == nb - 1)
            def _():
                cat = jnp.concatenate([vn_ref[...], acc_ref[...]], axis=1)
                sh_ref[...] = lax.dot_general(
                    cat, W3_ref[...], (((1,), (1,)), ((), ())),
                    preferred_element_type=jnp.float32) + b3_ref[...]

        @pl.when(i >= nb)
        def _vocab():
            z_ref[...] = lax.dot_general(sh_ref[...], iw_ref[...],
                                         (((1,), (1,)), ((), ())),
                                         preferred_element_type=jnp.float32)

    return _body


def kernel(session_embedding, batch, item_weight, W1, b1, W2, b2, Wq, bq, W3, b3):
    T, H = session_embedding.shape
    V = item_weight.shape[0]
    batch32 = batch.astype(jnp.int32)

    nb = T // _TB
    nv = pl.cdiv(V, _VB)
    batch2 = batch32.reshape(T // 128, 128)
    batch3 = batch32.reshape(nb, 1, _TB)
    bias = (b1 + b2).reshape(1, H)
    Wqb = jnp.broadcast_to(Wq, (_B, H))

    tok = lambda i: jnp.minimum(i, nb - 1)
    voc = lambda i: jnp.maximum(i - nb, 0)

    z = pl.pallas_call(
        _make_body(nb),
        grid=(nb + nv,),
        in_specs=[
            pl.BlockSpec(memory_space=pltpu.SMEM),
            pl.BlockSpec((T // 128, 128), lambda i: (0, 0)),
            pl.BlockSpec((1, 1, _TB), lambda i: (tok(i), 0, 0)),
            pl.BlockSpec(memory_space=pl.ANY),
            pl.BlockSpec((_TB, H), lambda i: (tok(i), 0)),
            pl.BlockSpec((H, H), lambda i: (0, 0)),
            pl.BlockSpec((H, H), lambda i: (0, 0)),
            pl.BlockSpec((_B, H), lambda i: (0, 0)),
            pl.BlockSpec((1, H), lambda i: (0, 0)),
            pl.BlockSpec((H, 2 * H), lambda i: (0, 0)),
            pl.BlockSpec((1, H), lambda i: (0, 0)),
            pl.BlockSpec((_VB, H), lambda i: (voc(i), 0)),
        ],
        out_specs=pl.BlockSpec((_B, _VB), lambda i: (0, voc(i))),
        out_shape=jax.ShapeDtypeStruct((_B, V), jnp.float32),
        scratch_shapes=[pltpu.VMEM((_B, H), jnp.float32),
                        pltpu.VMEM((_B, H), jnp.float32),
                        pltpu.VMEM((_B, H), jnp.float32),
                        pltpu.SemaphoreType.DMA],
    )(bq, batch2, batch3, session_embedding, session_embedding, W1, W2,
      Wqb, bias, W3, b3.reshape(1, H), item_weight)
    return z


# speedup vs baseline: 1.0698x; 1.0024x over previous
import jax
import jax.numpy as jnp
from jax import lax
from jax.experimental import pallas as pl
from jax.experimental.pallas import tpu as pltpu

_B = 16
_TB = 4096
_VB = 10240


def _make_body(nb):
    def _body(bq_ref, batch2_ref, batch_ref, se_hbm, se_ref, W1_ref, W2_ref,
              Wqb_ref, bias_ref, W3_ref, b3_ref, iw_ref, z_ref,
              acc_ref, sh_ref, vn_ref, sem):
        i = pl.program_id(0)

        @pl.when(i == 0)
        def _head():
            b2d = batch2_ref[...]
            copies = []
            for s in range(_B):
                cnt = jnp.sum((b2d <= s).astype(jnp.int32))
                idx = jnp.maximum(cnt - 1, 0)
                cp = pltpu.make_async_copy(
                    se_hbm.at[pl.ds(idx, 1)], vn_ref.at[pl.ds(s, 1)], sem)
                cp.start()
                copies.append(cp)
            for cp in copies:
                cp.wait()

        @pl.when(i < nb)
        def _token():
            se = se_ref[...]
            tb = se.shape[0]
            brow = batch_ref[0]
            seg_iota = lax.broadcasted_iota(jnp.int32, (_B, tb), 0)
            ohT = (jnp.broadcast_to(brow, (_B, tb)) == seg_iota
                   ).astype(jnp.bfloat16)
            c = lax.dot_general(vn_ref[...], W1_ref[...],
                                (((1,), (1,)), ((), ())),
                                preferred_element_type=jnp.float32)
            c = (c + bias_ref[...]).astype(jnp.bfloat16)
            m = lax.dot_general(se, W2_ref[...], (((1,), (1,)), ((), ())),
                                preferred_element_type=jnp.float32)
            g = lax.dot_general(ohT, c, (((0,), (0,)), ((), ())),
                                preferred_element_type=jnp.float32)
            h = (0.5 + 0.5 * jnp.tanh(0.5 * (m + g))).astype(jnp.bfloat16)
            alphaT = lax.dot_general(Wqb_ref[...], h, (((1,), (1,)), ((), ())),
                                     preferred_element_type=jnp.float32)
            wT = (ohT.astype(jnp.float32) * (alphaT + bq_ref[0])
                  ).astype(jnp.bfloat16)
            part = lax.dot_general(wT, se.astype(jnp.bfloat16),
                                   (((1,), (0,)), ((), ())),
                                   preferred_element_type=jnp.float32)

            @pl.when(i == 0)
            def _():
                acc_ref[...] = jnp.zeros_like(acc_ref)

            acc_ref[...] += part

            @pl.when(i == nb - 1)
            def _():
                cat = jnp.concatenate([vn_ref[...], acc_ref[...]], axis=1)
                sh_ref[...] = lax.dot_general(
                    cat, W3_ref[...], (((1,), (1,)), ((), ())),
                    preferred_element_type=jnp.float32) + b3_ref[...]

        @pl.when(i >= nb)
        def _vocab():
            z_ref[...] = lax.dot_general(sh_ref[...], iw_ref[...],
                                         (((1,), (1,)), ((), ())),
                                         preferred_element_type=jnp.float32)

    return _body


def kernel(session_embedding, batch, item_weight, W1, b1, W2, b2, Wq, bq, W3, b3):
    T, H = session_embedding.shape
    V = item_weight.shape[0]
    batch32 = batch.astype(jnp.int32)

    nb = T // _TB
    nv = pl.cdiv(V, _VB)
    batch2 = batch32.reshape(T // 128, 128)
    batch3 = batch32.reshape(nb, 1, _TB)
    bias = (b1 + b2).reshape(1, H)
    Wqb = jnp.broadcast_to(Wq, (_B, H))

    tok = lambda i: jnp.minimum(i, nb - 1)
    voc = lambda i: jnp.maximum(i - nb, 0)

    z = pl.pallas_call(
        _make_body(nb),
        grid=(nb + nv,),
        in_specs=[
            pl.BlockSpec(memory_space=pltpu.SMEM),
            pl.BlockSpec((T // 128, 128), lambda i: (0, 0)),
            pl.BlockSpec((1, 1, _TB), lambda i: (tok(i), 0, 0)),
            pl.BlockSpec(memory_space=pl.ANY),
            pl.BlockSpec((_TB, H), lambda i: (tok(i), 0)),
            pl.BlockSpec((H, H), lambda i: (0, 0)),
            pl.BlockSpec((H, H), lambda i: (0, 0)),
            pl.BlockSpec((_B, H), lambda i: (0, 0)),
            pl.BlockSpec((1, H), lambda i: (0, 0)),
            pl.BlockSpec((H, 2 * H), lambda i: (0, 0)),
            pl.BlockSpec((1, H), lambda i: (0, 0)),
            pl.BlockSpec((_VB, H), lambda i: (voc(i), 0)),
        ],
        out_specs=pl.BlockSpec((_B, _VB), lambda i: (0, voc(i))),
        out_shape=jax.ShapeDtypeStruct((_B, V), jnp.float32),
        scratch_shapes=[pltpu.VMEM((_B, H), jnp.float32),
                        pltpu.VMEM((_B, H), jnp.float32),
                        pltpu.VMEM((_B, H), jnp.float32),
                        pltpu.SemaphoreType.DMA],
    )(bq, batch2, batch3, session_embedding, session_embedding, W1, W2,
      Wqb, bias, W3, b3.reshape(1, H), item_weight)
    return z
